# baseline (speedup 1.0000x reference)
import functools

import jax
import jax.numpy as jnp
import numpy as np
from jax import lax
from jax.experimental import pallas as pl
from jax.experimental.pallas import tpu as pltpu
from jax.experimental.pallas import tpu_sc as plsc

B = 8
N = 4096
S = 512
K = 32
DPAD = 128
R2 = np.float32(0.2 ** 2)


def _fps_body(planes_ref, out_ref, dist_ref):
    X = planes_ref[0]
    Y = planes_ref[1]
    Z = planes_ref[2]
    iota = lax.broadcasted_iota(jnp.int32, (B, N), 1)
    dist_ref[...] = jnp.full((B, N), 1e10, jnp.float32)

    def body(i, far):
        sel = iota == far
        cx = jnp.sum(jnp.where(sel, X, 0.0), axis=1, keepdims=True)
        cy = jnp.sum(jnp.where(sel, Y, 0.0), axis=1, keepdims=True)
        cz = jnp.sum(jnp.where(sel, Z, 0.0), axis=1, keepdims=True)
        out_ref[pl.ds(i, 1), :, :] = jnp.concatenate(
            [cx, cy, cz], axis=1)[None]
        dx = X - cx
        dy = Y - cy
        dz = Z - cz
        sx = dx * dx
        sy = dy * dy
        sz = dz * dz
        d = (sx + sy) + sz
        dist = jnp.minimum(dist_ref[...], d)
        dist_ref[...] = dist
        m = jnp.max(dist, axis=1, keepdims=True)
        cand = jnp.where(dist == m, iota, N)
        return jnp.min(cand, axis=1, keepdims=True)

    lax.fori_loop(0, S, body, jnp.zeros((B, 1), jnp.int32))


def _fps(planes):
    return pl.pallas_call(
        _fps_body,
        out_shape=jax.ShapeDtypeStruct((S, B, 3), jnp.float32),
        scratch_shapes=[pltpu.VMEM((B, N), jnp.float32)],
    )(planes)


def _take16(v, idx):
    return lax.gather(
        v, idx[:, None],
        dimension_numbers=lax.GatherDimensionNumbers(
            offset_dims=(), collapsed_slice_dims=(0,), start_index_map=(0,)),
        slice_sizes=(1,),
        mode=lax.GatherScatterMode.PROMISE_IN_BOUNDS)


def _make_sc_gather():
    info = plsc.get_sparse_core_info()
    NC, NS = info.num_cores, info.num_subcores
    NW = NC * NS
    CPW = (B * S) // NW
    TPB = NW // B
    mesh = plsc.VectorSubcoreMesh(core_axis_name="c", subcore_axis_name="s")

    @functools.partial(
        pl.kernel,
        mesh=mesh,
        out_type=jax.ShapeDtypeStruct((B * S * K, DPAD), jnp.float32),
        scratch_types=[
            pltpu.VMEM((3 * N,), jnp.float32),
            pltpu.VMEM((16 * S,), jnp.float32),
            pltpu.VMEM((64,), jnp.int32),
            pltpu.VMEM((K,), jnp.int32),
            pltpu.VMEM((K,), jnp.int32),
            pltpu.VMEM((K, DPAD), jnp.float32),
            pltpu.VMEM((K, DPAD), jnp.float32),
            pltpu.SemaphoreType.DMA,
            pltpu.SemaphoreType.DMA,
            pltpu.SemaphoreType.DMA,
            pltpu.SemaphoreType.DMA,
        ],
        compiler_params=pltpu.CompilerParams(needs_layout_passes=False),
    )
    def sck(xyzT_hbm, cT_hbm, table_hbm, out_hbm,
            planes_v, cents_v, idxbuf_v, idx0, idx1, rows0, rows1,
            gsem0, gsem1, osem0, osem1):
        wid = lax.axis_index("c") * NS + lax.axis_index("s")
        b = wid // TPB
        q = wid % TPB
        pltpu.sync_copy(xyzT_hbm.at[b], planes_v)
        pltpu.sync_copy(cT_hbm.at[b], cents_v)
        iota16 = lax.iota(jnp.int32, 16)
        zeros16 = jnp.zeros((16,), jnp.int32)
        base_pt = b * N

        def do_scan(i, idx_ref):
            sidx = q * CPW + i
            cv = cents_v[pl.ds(sidx * 16, 16)]
            cx = _take16(cv, zeros16)
            cy = _take16(cv, zeros16 + 1)
            cz = _take16(cv, zeros16 + 2)

            def chunk(ch, off):
                px = planes_v[pl.ds(ch * 16, 16)]
                py = planes_v[pl.ds(N + ch * 16, 16)]
                pz = planes_v[pl.ds(2 * N + ch * 16, 16)]
                dx = cx - px
                dy = cy - py
                dz = cz - pz
                sx = dx * dx
                sy = dy * dy
                sz = dz * dz
                d = (sx + sy) + sz
                msk = d <= R2
                c = msk.astype(jnp.int32)
                for sh in (1, 2, 4, 8):
                    shifted = _take16(c, jnp.maximum(iota16 - sh, 0))
                    c = c + jnp.where(iota16 >= sh, shifted, 0)
                pos = jnp.minimum(c + (off - 1), jnp.int32(48))
                plsc.store_scatter(idxbuf_v, [pos], iota16 + ch * 16,
                                   mask=msk)
                return off + c[15]

            BLK = 32

            def blk_body(bi, off):
                def work(off):
                    base = bi * BLK
                    return lax.fori_loop(
                        0, BLK, lambda j, o: chunk(base + j, o), off)
                return lax.cond(off < K, work, lambda o: o, off)

            off = lax.fori_loop(0, (N // 16) // BLK, blk_body, jnp.int32(0))
            off = jnp.minimum(off, jnp.int32(K))
            v0 = idxbuf_v[pl.ds(0, 16)]
            v1 = idxbuf_v[pl.ds(16, 16)]
            first = _take16(v0, zeros16)
            offv = jnp.full((16,), off, jnp.int32)
            v0 = jnp.where(iota16 < offv, v0, first)
            v1 = jnp.where(iota16 + 16 < offv, v1, first)
            idx_ref[pl.ds(0, 16)] = v0 + base_pt
            idx_ref[pl.ds(16, 16)] = v1 + base_pt

        def out_view(i):
            return out_hbm.at[pl.ds((wid * CPW + i) * K, K)]

        def gstart(idx_ref, rows_ref, gsem):
            pltpu.async_copy(table_hbm.at[idx_ref], rows_ref, gsem)

        def gwait(idx_ref, rows_ref, gsem):
            pltpu.make_async_copy(table_hbm.at[idx_ref], rows_ref,
                                  gsem).wait()

        def ostart(rows_ref, i, osem):
            pltpu.async_copy(rows_ref, out_view(i), osem)

        def owait(rows_ref, i, osem):
            pltpu.make_async_copy(rows_ref, out_view(i), osem).wait()

        def pair(t, carry):
            i0 = 2 * t
            i1 = i0 + 1
            do_scan(i0, idx0)

            @pl.when(t > 0)
            def _():
                owait(rows0, i0 - 2, osem0)
            gstart(idx0, rows0, gsem0)

            @pl.when(t > 0)
            def _():
                gwait(idx1, rows1, gsem1)
                ostart(rows1, i0 - 1, osem1)

            do_scan(i1, idx1)

            @pl.when(t > 0)
            def _():
                owait(rows1, i1 - 2, osem1)
            gstart(idx1, rows1, gsem1)
            gwait(idx0, rows0, gsem0)
            ostart(rows0, i0, osem0)
            return carry

        lax.fori_loop(0, CPW // 2, pair, 0)
        gwait(idx1, rows1, gsem1)
        ostart(rows1, CPW - 1, osem1)
        owait(rows0, CPW - 2, osem0)
        owait(rows1, CPW - 1, osem1)

    return sck


_MLP_TILE = 64


def _mlp_body(g_ref, c_ref, w1_ref, b1_ref, w2_ref, b2_ref, w3_ref, b3_ref,
              o_ref):
    g = g_ref[...]
    c = c_ref[...]
    y = jnp.dot(g, w1_ref[...], preferred_element_type=jnp.float32)
    w1x = w1_ref[64:67, :]
    corr = (c[:, 0:1] * w1x[0:1, :] + c[:, 1:2] * w1x[1:2, :]
            + c[:, 2:3] * w1x[2:3, :])
    y = jax.nn.relu(y - corr + b1_ref[...])
    y = jax.nn.relu(jnp.dot(y, w2_ref[...], preferred_element_type=jnp.float32)
                    + b2_ref[...])
    y = jax.nn.relu(jnp.dot(y, w3_ref[...], preferred_element_type=jnp.float32)
                    + b3_ref[...])
    o_ref[...] = jnp.max(y.reshape(_MLP_TILE, K, 128), axis=1)


def _mlp(gathered, cexp, w1p, b1p, w2p, b2p, w3p, b3p):
    nsteps = (B * S) // _MLP_TILE
    rows = _MLP_TILE * K
    return pl.pallas_call(
        _mlp_body,
        grid=(nsteps,),
        in_specs=[
            pl.BlockSpec((rows, DPAD), lambda i: (i, 0)),
            pl.BlockSpec((rows, 3), lambda i: (i, 0)),
            pl.BlockSpec((DPAD, 64), lambda i: (0, 0)),
            pl.BlockSpec((1, 64), lambda i: (0, 0)),
            pl.BlockSpec((64, 64), lambda i: (0, 0)),
            pl.BlockSpec((1, 64), lambda i: (0, 0)),
            pl.BlockSpec((64, 128), lambda i: (0, 0)),
            pl.BlockSpec((1, 128), lambda i: (0, 0)),
        ],
        out_specs=pl.BlockSpec((_MLP_TILE, 128), lambda i: (i, 0)),
        out_shape=jax.ShapeDtypeStruct((B * S, 128), jnp.float32),
    )(gathered, cexp, w1p, b1p, w2p, b2p, w3p, b3p)


def kernel(x, xyz, W1, b1, g1, be1, W2, b2, g2, be2, W3, b3, g3, be3):
    scale = np.float32(1.0 / np.sqrt(1.0 + 1e-3))
    w1f = W1 * (scale * g1)[None, :]
    w1p = jnp.zeros((DPAD, 64), jnp.float32).at[:67, :].set(w1f)
    b1p = (b1 * scale * g1 + be1)[None, :]
    w2p = W2 * (scale * g2)[None, :]
    b2p = (b2 * scale * g2 + be2)[None, :]
    w3p = W3 * (scale * g3)[None, :]
    b3p = (b3 * scale * g3 + be3)[None, :]

    planes = xyz.transpose(2, 0, 1)

    newSB3 = _fps(planes)
    new_xyz = newSB3.transpose(1, 0, 2)
    cT = jnp.pad(new_xyz, ((0, 0), (0, 0), (0, 13))).reshape(B, 16 * S)

    xyzT = xyz.transpose(0, 2, 1).reshape(B, 3 * N)
    table = jnp.concatenate(
        [x, xyz, jnp.zeros((B, N, DPAD - 67), jnp.float32)], axis=-1
    ).reshape(B * N, DPAD)

    gathered = _make_sc_gather()(xyzT, cT, table)

    cexp = jnp.repeat(new_xyz.reshape(B * S, 3), K, axis=0)
    pooled = _mlp(gathered, cexp, w1p, b1p, w2p, b2p, w3p, b3p)
    return (new_xyz, pooled.reshape(B, S, 128))


# speedup vs baseline: 1.0255x; 1.0255x over previous
import functools

import jax
import jax.numpy as jnp
import numpy as np
from jax import lax
from jax.experimental import pallas as pl
from jax.experimental.pallas import tpu as pltpu
from jax.experimental.pallas import tpu_sc as plsc

B = 8
N = 4096
S = 512
K = 32
DPAD = 128
R2 = np.float32(0.2 ** 2)


def _fps_body(planes_ref, out_ref, dist_ref):
    X = planes_ref[0]
    Y = planes_ref[1]
    Z = planes_ref[2]
    iota = lax.broadcasted_iota(jnp.int32, (B, N), 1)
    dist_ref[...] = jnp.full((B, N), 1e10, jnp.float32)

    def body(i, far):
        sel = iota == far
        cx = jnp.sum(jnp.where(sel, X, 0.0), axis=1, keepdims=True)
        cy = jnp.sum(jnp.where(sel, Y, 0.0), axis=1, keepdims=True)
        cz = jnp.sum(jnp.where(sel, Z, 0.0), axis=1, keepdims=True)
        out_ref[pl.ds(i, 1), :, :] = jnp.concatenate(
            [cx, cy, cz], axis=1)[None]
        dx = X - cx
        dy = Y - cy
        dz = Z - cz
        sx = dx * dx
        sy = dy * dy
        sz = dz * dz
        d = (sx + sy) + sz
        dist = jnp.minimum(dist_ref[...], d)
        dist_ref[...] = dist
        m = jnp.max(dist, axis=1, keepdims=True)
        cand = jnp.where(dist == m, iota, N)
        return jnp.min(cand, axis=1, keepdims=True)

    lax.fori_loop(0, S, body, jnp.zeros((B, 1), jnp.int32))


def _fps(planes):
    return pl.pallas_call(
        _fps_body,
        out_shape=jax.ShapeDtypeStruct((S, B, 3), jnp.float32),
        scratch_shapes=[pltpu.VMEM((B, N), jnp.float32)],
    )(planes)


def _take16(v, idx):
    return lax.gather(
        v, idx[:, None],
        dimension_numbers=lax.GatherDimensionNumbers(
            offset_dims=(), collapsed_slice_dims=(0,), start_index_map=(0,)),
        slice_sizes=(1,),
        mode=lax.GatherScatterMode.PROMISE_IN_BOUNDS)


def _make_sc_gather():
    info = plsc.get_sparse_core_info()
    NC, NS = info.num_cores, info.num_subcores
    NW = NC * NS
    CPW = (B * S) // NW
    TPB = NW // B
    mesh = plsc.VectorSubcoreMesh(core_axis_name="c", subcore_axis_name="s")

    @functools.partial(
        pl.kernel,
        mesh=mesh,
        out_type=jax.ShapeDtypeStruct((B * S * K, DPAD), jnp.float32),
        scratch_types=[
            pltpu.VMEM((3 * N,), jnp.float32),
            pltpu.VMEM((16 * S,), jnp.float32),
            pltpu.VMEM((64,), jnp.int32),
            pltpu.VMEM((K,), jnp.int32),
            pltpu.VMEM((K,), jnp.int32),
            pltpu.VMEM((K, DPAD), jnp.float32),
            pltpu.VMEM((K, DPAD), jnp.float32),
            pltpu.SemaphoreType.DMA,
            pltpu.SemaphoreType.DMA,
            pltpu.SemaphoreType.DMA,
            pltpu.SemaphoreType.DMA,
        ],
        compiler_params=pltpu.CompilerParams(needs_layout_passes=False),
    )
    def sck(xyzT_hbm, cT_hbm, table_hbm, out_hbm,
            planes_v, cents_v, idxbuf_v, idx0, idx1, rows0, rows1,
            gsem0, gsem1, osem0, osem1):
        wid = lax.axis_index("c") * NS + lax.axis_index("s")
        b = wid // TPB
        q = wid % TPB
        pltpu.sync_copy(xyzT_hbm.at[b], planes_v)
        pltpu.sync_copy(cT_hbm.at[b], cents_v)
        iota16 = lax.iota(jnp.int32, 16)
        zeros16 = jnp.zeros((16,), jnp.int32)
        base_pt = b * N

        def do_scan(i, idx_ref):
            sidx = q * CPW + i
            cv = cents_v[pl.ds(sidx * 16, 16)]
            cx = _take16(cv, zeros16)
            cy = _take16(cv, zeros16 + 1)
            cz = _take16(cv, zeros16 + 2)

            def chunk(ch, off):
                px = planes_v[pl.ds(ch * 16, 16)]
                py = planes_v[pl.ds(N + ch * 16, 16)]
                pz = planes_v[pl.ds(2 * N + ch * 16, 16)]
                dx = cx - px
                dy = cy - py
                dz = cz - pz
                sx = dx * dx
                sy = dy * dy
                sz = dz * dz
                d = (sx + sy) + sz
                msk = d <= R2
                c = msk.astype(jnp.int32)
                for sh in (1, 2, 4, 8):
                    shifted = _take16(c, jnp.maximum(iota16 - sh, 0))
                    c = c + jnp.where(iota16 >= sh, shifted, 0)
                pos = jnp.minimum(c + (off - 1), jnp.int32(48))
                plsc.store_scatter(idxbuf_v, [pos], iota16 + ch * 16,
                                   mask=msk)
                return off + c[15]

            BLK = 8

            def blk_body(bi, off):
                def work(off):
                    base = bi * BLK
                    return lax.fori_loop(
                        0, BLK, lambda j, o: chunk(base + j, o), off)
                return lax.cond(off < K, work, lambda o: o, off)

            off = lax.fori_loop(0, (N // 16) // BLK, blk_body, jnp.int32(0))
            off = jnp.minimum(off, jnp.int32(K))
            v0 = idxbuf_v[pl.ds(0, 16)]
            v1 = idxbuf_v[pl.ds(16, 16)]
            first = _take16(v0, zeros16)
            offv = jnp.full((16,), off, jnp.int32)
            v0 = jnp.where(iota16 < offv, v0, first)
            v1 = jnp.where(iota16 + 16 < offv, v1, first)
            idx_ref[pl.ds(0, 16)] = v0 + base_pt
            idx_ref[pl.ds(16, 16)] = v1 + base_pt

        def out_view(i):
            return out_hbm.at[pl.ds((wid * CPW + i) * K, K)]

        def gstart(idx_ref, rows_ref, gsem):
            pltpu.async_copy(table_hbm.at[idx_ref], rows_ref, gsem)

        def gwait(idx_ref, rows_ref, gsem):
            pltpu.make_async_copy(table_hbm.at[idx_ref], rows_ref,
                                  gsem).wait()

        def ostart(rows_ref, i, osem):
            pltpu.async_copy(rows_ref, out_view(i), osem)

        def owait(rows_ref, i, osem):
            pltpu.make_async_copy(rows_ref, out_view(i), osem).wait()

        def pair(t, carry):
            i0 = 2 * t
            i1 = i0 + 1
            do_scan(i0, idx0)

            @pl.when(t > 0)
            def _():
                owait(rows0, i0 - 2, osem0)
            gstart(idx0, rows0, gsem0)

            @pl.when(t > 0)
            def _():
                gwait(idx1, rows1, gsem1)
                ostart(rows1, i0 - 1, osem1)

            do_scan(i1, idx1)

            @pl.when(t > 0)
            def _():
                owait(rows1, i1 - 2, osem1)
            gstart(idx1, rows1, gsem1)
            gwait(idx0, rows0, gsem0)
            ostart(rows0, i0, osem0)
            return carry

        lax.fori_loop(0, CPW // 2, pair, 0)
        gwait(idx1, rows1, gsem1)
        ostart(rows1, CPW - 1, osem1)
        owait(rows0, CPW - 2, osem0)
        owait(rows1, CPW - 1, osem1)

    return sck


_MLP_TILE = 64


def _mlp_body(g_ref, c_ref, w1_ref, b1_ref, w2_ref, b2_ref, w3_ref, b3_ref,
              o_ref):
    g = g_ref[...]
    c = c_ref[...]
    y = jnp.dot(g, w1_ref[...], preferred_element_type=jnp.float32)
    w1x = w1_ref[64:67, :]
    corr = (c[:, 0:1] * w1x[0:1, :] + c[:, 1:2] * w1x[1:2, :]
            + c[:, 2:3] * w1x[2:3, :])
    y = jax.nn.relu(y - corr + b1_ref[...])
    y = jax.nn.relu(jnp.dot(y, w2_ref[...], preferred_element_type=jnp.float32)
                    + b2_ref[...])
    y = jax.nn.relu(jnp.dot(y, w3_ref[...], preferred_element_type=jnp.float32)
                    + b3_ref[...])
    o_ref[...] = jnp.max(y.reshape(_MLP_TILE, K, 128), axis=1)


def _mlp(gathered, cexp, w1p, b1p, w2p, b2p, w3p, b3p):
    nsteps = (B * S) // _MLP_TILE
    rows = _MLP_TILE * K
    return pl.pallas_call(
        _mlp_body,
        grid=(nsteps,),
        in_specs=[
            pl.BlockSpec((rows, DPAD), lambda i: (i, 0)),
            pl.BlockSpec((rows, 3), lambda i: (i, 0)),
            pl.BlockSpec((DPAD, 64), lambda i: (0, 0)),
            pl.BlockSpec((1, 64), lambda i: (0, 0)),
            pl.BlockSpec((64, 64), lambda i: (0, 0)),
            pl.BlockSpec((1, 64), lambda i: (0, 0)),
            pl.BlockSpec((64, 128), lambda i: (0, 0)),
            pl.BlockSpec((1, 128), lambda i: (0, 0)),
        ],
        out_specs=pl.BlockSpec((_MLP_TILE, 128), lambda i: (i, 0)),
        out_shape=jax.ShapeDtypeStruct((B * S, 128), jnp.float32),
    )(gathered, cexp, w1p, b1p, w2p, b2p, w3p, b3p)


def kernel(x, xyz, W1, b1, g1, be1, W2, b2, g2, be2, W3, b3, g3, be3):
    scale = np.float32(1.0 / np.sqrt(1.0 + 1e-3))
    w1f = W1 * (scale * g1)[None, :]
    w1p = jnp.zeros((DPAD, 64), jnp.float32).at[:67, :].set(w1f)
    b1p = (b1 * scale * g1 + be1)[None, :]
    w2p = W2 * (scale * g2)[None, :]
    b2p = (b2 * scale * g2 + be2)[None, :]
    w3p = W3 * (scale * g3)[None, :]
    b3p = (b3 * scale * g3 + be3)[None, :]

    planes = xyz.transpose(2, 0, 1)

    newSB3 = _fps(planes)
    new_xyz = newSB3.transpose(1, 0, 2)
    cT = jnp.pad(new_xyz, ((0, 0), (0, 0), (0, 13))).reshape(B, 16 * S)

    xyzT = xyz.transpose(0, 2, 1).reshape(B, 3 * N)
    table = jnp.concatenate(
        [x, xyz, jnp.zeros((B, N, DPAD - 67), jnp.float32)], axis=-1
    ).reshape(B * N, DPAD)

    gathered = _make_sc_gather()(xyzT, cT, table)

    cexp = jnp.repeat(new_xyz.reshape(B * S, 3), K, axis=0)
    pooled = _mlp(gathered, cexp, w1p, b1p, w2p, b2p, w3p, b3p)
    return (new_xyz, pooled.reshape(B, S, 128))
